# baseline (device time: 406269 ns/iter reference)
import jax
import jax.numpy as jnp
from jax import lax
from jax.experimental import pallas as pl
from jax.experimental.pallas import tpu as pltpu

M = 4096
K = 8192
N = 4096
MQ = M // 4
N_BLK = 256
K_BLK = 2048
N_STEPS = N // N_BLK
K_STEPS = K // K_BLK


def kernel(dy, W):
    my_x = lax.axis_index("x")
    my_y = lax.axis_index("y")
    q = 2 * my_x + my_y

    def body(q_ref, dy_ref, w_ref, out_ref, c_ref, zrecv_ref,
             local_sem, z_send, z_recv, xd_send, xd_recv,
             yd_send, yd_recv, yf_send, yf_recv, xf_send, xf_recv,
             zf_send, zf_recv):
        k = pl.program_id(0)
        n = pl.program_id(1)
        mx = lax.axis_index("x")
        my = lax.axis_index("y")
        mz = lax.axis_index("z")
        qq = 2 * mx + my
        q1 = 2 * mx + (1 - my)
        q2 = 2 * (1 - mx) + my
        q3 = 2 * (1 - mx) + (1 - my)
        rows_q = pl.ds(qq * MQ, MQ)
        rows_q1 = pl.ds(q1 * MQ, MQ)
        rows_q2 = pl.ds(q2 * MQ, MQ)
        rows_q3 = pl.ds(q3 * MQ, MQ)
        HF = N_BLK // 2
        yf_off = mz * HF

        def csl(m):
            return pl.ds(m * N_BLK, N_BLK)

        def z_descr(m):
            return pltpu.make_async_remote_copy(
                src_ref=c_ref.at[:, csl(m)], dst_ref=zrecv_ref.at[:, csl(m)],
                send_sem=z_send.at[m], recv_sem=z_recv.at[m],
                device_id=(mx, my, 1 - mz),
                device_id_type=pl.DeviceIdType.MESH,
            )

        def local_descr(m):
            return pltpu.make_async_copy(
                c_ref.at[:, csl(m)], out_ref.at[rows_q, csl(m)],
                local_sem.at[m],
            )

        def xd_descr(m):
            return pltpu.make_async_remote_copy(
                src_ref=c_ref.at[:, csl(m)], dst_ref=out_ref.at[rows_q, csl(m)],
                send_sem=xd_send.at[m], recv_sem=xd_recv.at[m],
                device_id=(1 - mx, my, mz),
                device_id_type=pl.DeviceIdType.MESH,
            )

        def yd_descr(m):
            return pltpu.make_async_remote_copy(
                src_ref=c_ref.at[:, csl(m)], dst_ref=out_ref.at[rows_q, csl(m)],
                send_sem=yd_send.at[m], recv_sem=yd_recv.at[m],
                device_id=(mx, 1 - my, mz),
                device_id_type=pl.DeviceIdType.MESH,
            )

        def yf_descr(m):
            sl = pl.ds(m * N_BLK + yf_off, HF)
            return pltpu.make_async_remote_copy(
                src_ref=out_ref.at[rows_q2, sl],
                dst_ref=out_ref.at[rows_q2, sl],
                send_sem=yf_send.at[m], recv_sem=yf_recv.at[m],
                device_id=(mx, 1 - my, mz),
                device_id_type=pl.DeviceIdType.MESH,
            )

        def xf_descr(m):
            sl = csl(m)
            return pltpu.make_async_remote_copy(
                src_ref=out_ref.at[rows_q1, sl],
                dst_ref=out_ref.at[rows_q1, sl],
                send_sem=xf_send.at[m], recv_sem=xf_recv.at[m],
                device_id=(1 - mx, my, mz),
                device_id_type=pl.DeviceIdType.MESH,
            )

        def zf_descr(m):
            sl = pl.ds(m * N_BLK + yf_off, HF)
            return pltpu.make_async_remote_copy(
                src_ref=out_ref.at[rows_q3, sl],
                dst_ref=out_ref.at[rows_q3, sl],
                send_sem=zf_send.at[m], recv_sem=zf_recv.at[m],
                device_id=(mx, my, 1 - mz),
                device_id_type=pl.DeviceIdType.MESH,
            )

        def handle_z(m):
            z_descr(m).wait()
            sl = csl(m)
            c_ref[:, sl] = c_ref[:, sl] + zrecv_ref[:, sl]
            local_descr(m).start()
            xd_descr(m).start()
            yd_descr(m).start()

        def handle_xy(m):
            xd_descr(m).wait()
            yd_descr(m).wait()

            @pl.when(m % 3 == 2)
            def _():
                xf_descr(m).start()

            @pl.when(m % 3 != 2)
            def _():
                yf_descr(m).start()

        def handle_zf(m):
            @pl.when(m % 3 != 2)
            def _():
                yf_descr(m).wait()
                zf_descr(m).start()

        @pl.when(jnp.logical_and(k == 0, n == 0))
        def _barrier():
            bsem = pltpu.get_barrier_semaphore()
            for nbr in ((1 - mx, my, mz), (mx, 1 - my, mz), (mx, my, 1 - mz)):
                pl.semaphore_signal(bsem, inc=1, device_id=nbr,
                                    device_id_type=pl.DeviceIdType.MESH)
            pl.semaphore_wait(bsem, 3)

        acc = lax.dot_general(
            dy_ref[...], w_ref[...],
            (((1,), (1,)), ((), ())),
            precision=lax.Precision.DEFAULT,
            preferred_element_type=jnp.float32,
        )
        nsl = csl(n)

        @pl.when(k == 0)
        def _init():
            c_ref[:, nsl] = acc

        @pl.when(k != 0)
        def _accum():
            c_ref[:, nsl] = c_ref[:, nsl] + acc

        @pl.when(k == K_STEPS - 1)
        def _comm():
            z_descr(n).start()

            @pl.when(n >= 1)
            def _():
                handle_z(n - 1)

            @pl.when(n >= 2)
            def _():
                handle_xy(n - 2)

            @pl.when(n >= 3)
            def _():
                handle_zf(n - 3)

            @pl.when(n == N_STEPS - 1)
            def _drain():
                handle_z(n)
                handle_xy(n - 1)
                handle_xy(n)
                handle_zf(n - 2)
                handle_zf(n - 1)
                handle_zf(n)
                for m in range(N_STEPS):
                    if m % 3 == 2:
                        xf_descr(m).wait()
                    else:
                        zf_descr(m).wait()
                    local_descr(m).wait()

    grid_spec = pltpu.PrefetchScalarGridSpec(
        num_scalar_prefetch=1,
        grid=(K_STEPS, N_STEPS),
        in_specs=[
            pl.BlockSpec((MQ, K_BLK), lambda k, n, qs: (qs[0], k)),
            pl.BlockSpec((N_BLK, K_BLK), lambda k, n, qs: (n, k)),
        ],
        out_specs=pl.BlockSpec(memory_space=pl.ANY),
        scratch_shapes=[
            pltpu.VMEM((MQ, N), jnp.float32),
            pltpu.VMEM((MQ, N), jnp.float32),
            pltpu.SemaphoreType.DMA((N_STEPS,)),
            pltpu.SemaphoreType.DMA((N_STEPS,)),
            pltpu.SemaphoreType.DMA((N_STEPS,)),
            pltpu.SemaphoreType.DMA((N_STEPS,)),
            pltpu.SemaphoreType.DMA((N_STEPS,)),
            pltpu.SemaphoreType.DMA((N_STEPS,)),
            pltpu.SemaphoreType.DMA((N_STEPS,)),
            pltpu.SemaphoreType.DMA((N_STEPS,)),
            pltpu.SemaphoreType.DMA((N_STEPS,)),
            pltpu.SemaphoreType.DMA((N_STEPS,)),
            pltpu.SemaphoreType.DMA((N_STEPS,)),
            pltpu.SemaphoreType.DMA((N_STEPS,)),
            pltpu.SemaphoreType.DMA((N_STEPS,)),
        ],
    )

    return pl.pallas_call(
        body,
        grid_spec=grid_spec,
        out_shape=jax.ShapeDtypeStruct((M, N), jnp.float32),
        compiler_params=pltpu.CompilerParams(
            collective_id=0,
            dimension_semantics=("arbitrary", "arbitrary"),
            vmem_limit_bytes=64 * 1024 * 1024,
        ),
    )(q[None].astype(jnp.int32), dy, W)


# device time: 348058 ns/iter; 1.1672x vs baseline; 1.1672x over previous
import jax
import jax.numpy as jnp
from jax import lax
from jax.experimental import pallas as pl
from jax.experimental.pallas import tpu as pltpu

M = 4096
K = 8192
N = 4096
MQ = M // 4
N_BLK = 256
K_BLK = 2048
N_STEPS = N // N_BLK
K_STEPS = K // K_BLK


def kernel(dy, W):
    my_x = lax.axis_index("x")
    my_y = lax.axis_index("y")
    q = 2 * my_x + my_y
    dy_qb = lax.dynamic_slice(dy, (q * MQ, 0), (MQ, K)).astype(jnp.bfloat16)

    def body(dy_ref, w_ref, out_ref, c_ref, zrecv_ref,
             local_sem, z_send, z_recv, xd_send, xd_recv,
             yd_send, yd_recv, yf_send, yf_recv, xf_send, xf_recv,
             zf_send, zf_recv):
        n = pl.program_id(0)
        k = pl.program_id(1)
        mx = lax.axis_index("x")
        my = lax.axis_index("y")
        mz = lax.axis_index("z")
        qq = 2 * mx + my
        q1 = 2 * mx + (1 - my)
        q2 = 2 * (1 - mx) + my
        q3 = 2 * (1 - mx) + (1 - my)
        rows_q = pl.ds(qq * MQ, MQ)
        rows_q1 = pl.ds(q1 * MQ, MQ)
        rows_q2 = pl.ds(q2 * MQ, MQ)
        rows_q3 = pl.ds(q3 * MQ, MQ)
        HF = N_BLK // 2
        yf_off = mz * HF

        def csl(m):
            return pl.ds(m * N_BLK, N_BLK)

        def z_descr(m):
            return pltpu.make_async_remote_copy(
                src_ref=c_ref.at[:, csl(m)], dst_ref=zrecv_ref.at[:, csl(m)],
                send_sem=z_send.at[m], recv_sem=z_recv.at[m],
                device_id=(mx, my, 1 - mz),
                device_id_type=pl.DeviceIdType.MESH,
            )

        def local_descr(m):
            return pltpu.make_async_copy(
                c_ref.at[:, csl(m)], out_ref.at[rows_q, csl(m)],
                local_sem.at[m],
            )

        def xd_descr(m):
            return pltpu.make_async_remote_copy(
                src_ref=c_ref.at[:, csl(m)], dst_ref=out_ref.at[rows_q, csl(m)],
                send_sem=xd_send.at[m], recv_sem=xd_recv.at[m],
                device_id=(1 - mx, my, mz),
                device_id_type=pl.DeviceIdType.MESH,
            )

        def yd_descr(m):
            return pltpu.make_async_remote_copy(
                src_ref=c_ref.at[:, csl(m)], dst_ref=out_ref.at[rows_q, csl(m)],
                send_sem=yd_send.at[m], recv_sem=yd_recv.at[m],
                device_id=(mx, 1 - my, mz),
                device_id_type=pl.DeviceIdType.MESH,
            )

        def yf_descr(m):
            sl = pl.ds(m * N_BLK + yf_off, HF)
            return pltpu.make_async_remote_copy(
                src_ref=out_ref.at[rows_q2, sl],
                dst_ref=out_ref.at[rows_q2, sl],
                send_sem=yf_send.at[m], recv_sem=yf_recv.at[m],
                device_id=(mx, 1 - my, mz),
                device_id_type=pl.DeviceIdType.MESH,
            )

        def xf_descr(m):
            sl = csl(m)
            return pltpu.make_async_remote_copy(
                src_ref=out_ref.at[rows_q1, sl],
                dst_ref=out_ref.at[rows_q1, sl],
                send_sem=xf_send.at[m], recv_sem=xf_recv.at[m],
                device_id=(1 - mx, my, mz),
                device_id_type=pl.DeviceIdType.MESH,
            )

        def zf_descr(m):
            sl = pl.ds(m * N_BLK + yf_off, HF)
            return pltpu.make_async_remote_copy(
                src_ref=out_ref.at[rows_q3, sl],
                dst_ref=out_ref.at[rows_q3, sl],
                send_sem=zf_send.at[m], recv_sem=zf_recv.at[m],
                device_id=(mx, my, 1 - mz),
                device_id_type=pl.DeviceIdType.MESH,
            )

        def handle_z(m):
            z_descr(m).wait()
            sl = csl(m)
            c_ref[:, sl] = c_ref[:, sl] + zrecv_ref[:, sl]
            local_descr(m).start()
            xd_descr(m).start()
            yd_descr(m).start()

        def handle_xy(m):
            xd_descr(m).wait()
            yd_descr(m).wait()

            @pl.when(m % 3 == 2)
            def _():
                xf_descr(m).start()

            @pl.when(m % 3 != 2)
            def _():
                yf_descr(m).start()

        def handle_zf(m):
            @pl.when(m % 3 != 2)
            def _():
                yf_descr(m).wait()
                zf_descr(m).start()

        @pl.when(jnp.logical_and(n == 0, k == 0))
        def _barrier():
            bsem = pltpu.get_barrier_semaphore()
            for nbr in ((1 - mx, my, mz), (mx, 1 - my, mz), (mx, my, 1 - mz)):
                pl.semaphore_signal(bsem, inc=1, device_id=nbr,
                                    device_id_type=pl.DeviceIdType.MESH)
            pl.semaphore_wait(bsem, 3)

        acc = lax.dot_general(
            dy_ref[:, pl.ds(k * K_BLK, K_BLK)],
            w_ref[...].astype(jnp.bfloat16),
            (((1,), (1,)), ((), ())),
            preferred_element_type=jnp.float32,
        )
        nsl = csl(n)

        @pl.when(k == 0)
        def _init():
            c_ref[:, nsl] = acc

        @pl.when(k != 0)
        def _accum():
            c_ref[:, nsl] = c_ref[:, nsl] + acc

        @pl.when(k == K_STEPS - 1)
        def _comm():
            z_descr(n).start()

            @pl.when(n >= 1)
            def _():
                handle_z(n - 1)

            @pl.when(n >= 2)
            def _():
                handle_xy(n - 2)

            @pl.when(n >= 3)
            def _():
                handle_zf(n - 3)

            @pl.when(n == N_STEPS - 1)
            def _drain():
                handle_z(n)
                handle_xy(n - 1)
                handle_xy(n)
                handle_zf(n - 2)
                handle_zf(n - 1)
                handle_zf(n)
                for m in range(N_STEPS):
                    if m % 3 == 2:
                        xf_descr(m).wait()
                    else:
                        zf_descr(m).wait()
                    local_descr(m).wait()

    return pl.pallas_call(
        body,
        grid=(N_STEPS, K_STEPS),
        in_specs=[
            pl.BlockSpec(memory_space=pltpu.MemorySpace.VMEM),
            pl.BlockSpec((N_BLK, K_BLK), lambda n, k: (n, k)),
        ],
        out_specs=pl.BlockSpec(memory_space=pl.ANY),
        out_shape=jax.ShapeDtypeStruct((M, N), jnp.float32),
        scratch_shapes=[
            pltpu.VMEM((MQ, N), jnp.float32),
            pltpu.VMEM((MQ, N), jnp.float32),
            pltpu.SemaphoreType.DMA((N_STEPS,)),
            pltpu.SemaphoreType.DMA((N_STEPS,)),
            pltpu.SemaphoreType.DMA((N_STEPS,)),
            pltpu.SemaphoreType.DMA((N_STEPS,)),
            pltpu.SemaphoreType.DMA((N_STEPS,)),
            pltpu.SemaphoreType.DMA((N_STEPS,)),
            pltpu.SemaphoreType.DMA((N_STEPS,)),
            pltpu.SemaphoreType.DMA((N_STEPS,)),
            pltpu.SemaphoreType.DMA((N_STEPS,)),
            pltpu.SemaphoreType.DMA((N_STEPS,)),
            pltpu.SemaphoreType.DMA((N_STEPS,)),
            pltpu.SemaphoreType.DMA((N_STEPS,)),
            pltpu.SemaphoreType.DMA((N_STEPS,)),
        ],
        compiler_params=pltpu.CompilerParams(
            collective_id=0,
            dimension_semantics=("arbitrary", "arbitrary"),
            vmem_limit_bytes=64 * 1024 * 1024,
        ),
    )(dy_qb, W)


# device time: 245435 ns/iter; 1.6553x vs baseline; 1.4181x over previous
import jax
import jax.numpy as jnp
from jax import lax
from jax.experimental import pallas as pl
from jax.experimental.pallas import tpu as pltpu

M = 4096
K = 8192
N = 4096
MQ = M // 4
N_BLK = 256
K_BLK = 2048
N_STEPS = N // N_BLK
K_STEPS = K // K_BLK
NSLOT = 4
HF = N_BLK // 2
CHUNK = 1024
STAGE_SLOTS = 2


def kernel(dy, W):
    my_x = lax.axis_index("x")
    my_y = lax.axis_index("y")
    q = 2 * my_x + my_y
    dy_qb = lax.dynamic_slice(dy, (q * MQ, 0), (MQ, K)).astype(jnp.bfloat16)

    def body(dy_ref, w_ref, out_ref, c_ref, cb_ref, zr_ref, gb_ref,
             stage_ref, stage_sem, local_sem, z_send, z_recv,
             xd_send, xd_recv, yd_send, yd_recv, yf_send, yf_recv,
             xf_send, xf_recv, zf_send, zf_recv):
        n = pl.program_id(0)
        k = pl.program_id(1)
        mx = lax.axis_index("x")
        my = lax.axis_index("y")
        mz = lax.axis_index("z")
        qq = 2 * mx + my
        q1 = 2 * mx + (1 - my)
        q2 = 2 * (1 - mx) + my
        q3 = 2 * (1 - mx) + (1 - my)
        rows_q = pl.ds(qq * MQ, MQ)
        yf_off = mz * HF
        GB_X, GB_Y, GB_D = 0, N, 2 * N

        def csl(m):
            return pl.ds(m * N_BLK, N_BLK)

        def rsl(m):
            return pl.ds((m % NSLOT) * N_BLK, N_BLK)

        def z_descr(m):
            return pltpu.make_async_remote_copy(
                src_ref=cb_ref.at[:, rsl(m)], dst_ref=zr_ref.at[:, rsl(m)],
                send_sem=z_send.at[m], recv_sem=z_recv.at[m],
                device_id=(mx, my, 1 - mz),
                device_id_type=pl.DeviceIdType.MESH,
            )

        def local_descr(m):
            return pltpu.make_async_copy(
                c_ref.at[:, rsl(m)], out_ref.at[rows_q, csl(m)],
                local_sem.at[m],
            )

        def xd_descr(m):
            return pltpu.make_async_remote_copy(
                src_ref=cb_ref.at[:, rsl(m)],
                dst_ref=gb_ref.at[:, pl.ds(GB_X + m * N_BLK, N_BLK)],
                send_sem=xd_send.at[m], recv_sem=xd_recv.at[m],
                device_id=(1 - mx, my, mz),
                device_id_type=pl.DeviceIdType.MESH,
            )

        def yd_descr(m):
            return pltpu.make_async_remote_copy(
                src_ref=cb_ref.at[:, rsl(m)],
                dst_ref=gb_ref.at[:, pl.ds(GB_Y + m * N_BLK, N_BLK)],
                send_sem=yd_send.at[m], recv_sem=yd_recv.at[m],
                device_id=(mx, 1 - my, mz),
                device_id_type=pl.DeviceIdType.MESH,
            )

        def yf_descr(m):
            return pltpu.make_async_remote_copy(
                src_ref=gb_ref.at[:, pl.ds(GB_X + m * N_BLK + yf_off, HF)],
                dst_ref=gb_ref.at[:, pl.ds(GB_D + m * N_BLK + yf_off, HF)],
                send_sem=yf_send.at[m], recv_sem=yf_recv.at[m],
                device_id=(mx, 1 - my, mz),
                device_id_type=pl.DeviceIdType.MESH,
            )

        def xf_descr(m):
            return pltpu.make_async_remote_copy(
                src_ref=gb_ref.at[:, pl.ds(GB_Y + m * N_BLK, N_BLK)],
                dst_ref=gb_ref.at[:, pl.ds(GB_D + m * N_BLK, N_BLK)],
                send_sem=xf_send.at[m], recv_sem=xf_recv.at[m],
                device_id=(1 - mx, my, mz),
                device_id_type=pl.DeviceIdType.MESH,
            )

        def zf_descr(m):
            return pltpu.make_async_remote_copy(
                src_ref=gb_ref.at[:, pl.ds(GB_D + m * N_BLK + yf_off, HF)],
                dst_ref=gb_ref.at[:, pl.ds(GB_D + m * N_BLK + yf_off, HF)],
                send_sem=zf_send.at[m], recv_sem=zf_recv.at[m],
                device_id=(mx, my, 1 - mz),
                device_id_type=pl.DeviceIdType.MESH,
            )

        def handle_z(m):
            z_descr(m).wait()
            red = c_ref[:, rsl(m)] + zr_ref[:, rsl(m)].astype(jnp.float32)
            c_ref[:, rsl(m)] = red
            cb_ref[:, rsl(m)] = red.astype(jnp.bfloat16)
            local_descr(m).start()
            xd_descr(m).start()
            yd_descr(m).start()

        def handle_xy(m):
            xd_descr(m).wait()
            yd_descr(m).wait()
            local_descr(m).wait()

            @pl.when(m % 3 == 2)
            def _():
                xf_descr(m).start()

            @pl.when(m % 3 != 2)
            def _():
                yf_descr(m).start()

        def handle_zf(m):
            @pl.when(m % 3 != 2)
            def _():
                yf_descr(m).wait()
                zf_descr(m).start()

        @pl.when(jnp.logical_and(n == 0, k == 0))
        def _barrier():
            bsem = pltpu.get_barrier_semaphore()
            for nbr in ((1 - mx, my, mz), (mx, 1 - my, mz), (mx, my, 1 - mz)):
                pl.semaphore_signal(bsem, inc=1, device_id=nbr,
                                    device_id_type=pl.DeviceIdType.MESH)
            pl.semaphore_wait(bsem, 3)

        acc = lax.dot_general(
            dy_ref[:, pl.ds(k * K_BLK, K_BLK)],
            w_ref[...].astype(jnp.bfloat16),
            (((1,), (1,)), ((), ())),
            preferred_element_type=jnp.float32,
        )

        @pl.when(k == 0)
        def _init():
            c_ref[:, rsl(n)] = acc

        @pl.when(jnp.logical_and(k != 0, k != K_STEPS - 1))
        def _accum():
            c_ref[:, rsl(n)] = c_ref[:, rsl(n)] + acc

        @pl.when(k == K_STEPS - 1)
        def _comm():
            cfin = c_ref[:, rsl(n)] + acc
            c_ref[:, rsl(n)] = cfin
            cb_ref[:, rsl(n)] = cfin.astype(jnp.bfloat16)
            z_descr(n).start()

            @pl.when(n >= 1)
            def _():
                handle_z(n - 1)

            @pl.when(n >= 2)
            def _():
                handle_xy(n - 2)

            @pl.when(n >= 3)
            def _():
                handle_zf(n - 3)

            @pl.when(n == N_STEPS - 1)
            def _drain():
                handle_z(n)
                handle_xy(n - 1)
                handle_xy(n)
                handle_zf(n - 2)
                handle_zf(n - 1)
                handle_zf(n)
                for m in range(N_STEPS):
                    if m % 3 == 2:
                        xf_descr(m).wait()
                    else:
                        zf_descr(m).wait()
                def stage_descr(slot, quarter, ci):
                    return pltpu.make_async_copy(
                        stage_ref.at[:, pl.ds(slot * CHUNK, CHUNK)],
                        out_ref.at[pl.ds(quarter * MQ, MQ),
                                   pl.ds(ci * CHUNK, CHUNK)],
                        stage_sem.at[slot],
                    )

                streams = ((GB_X, q2), (GB_Y, q1), (GB_D, q3))
                prev = {}
                idx = 0
                for origin, quarter in streams:
                    for ci in range(N // CHUNK):
                        slot = idx % STAGE_SLOTS
                        if slot in prev:
                            stage_descr(slot, *prev[slot]).wait()
                        stage_ref[:, pl.ds(slot * CHUNK, CHUNK)] = gb_ref[
                            :, pl.ds(origin + ci * CHUNK, CHUNK)
                        ].astype(jnp.float32)
                        stage_descr(slot, quarter, ci).start()
                        prev[slot] = (quarter, ci)
                        idx += 1
                for slot in range(STAGE_SLOTS):
                    stage_descr(slot, *prev[slot]).wait()

    return pl.pallas_call(
        body,
        grid=(N_STEPS, K_STEPS),
        in_specs=[
            pl.BlockSpec(memory_space=pltpu.MemorySpace.VMEM),
            pl.BlockSpec((N_BLK, K_BLK), lambda n, k: (n, k)),
        ],
        out_specs=pl.BlockSpec(memory_space=pl.ANY),
        out_shape=jax.ShapeDtypeStruct((M, N), jnp.float32),
        scratch_shapes=[
            pltpu.VMEM((MQ, NSLOT * N_BLK), jnp.float32),
            pltpu.VMEM((MQ, NSLOT * N_BLK), jnp.bfloat16),
            pltpu.VMEM((MQ, NSLOT * N_BLK), jnp.bfloat16),
            pltpu.VMEM((MQ, 3 * N), jnp.bfloat16),
            pltpu.VMEM((MQ, STAGE_SLOTS * CHUNK), jnp.float32),
            pltpu.SemaphoreType.DMA((STAGE_SLOTS,)),
            pltpu.SemaphoreType.DMA((N_STEPS,)),
            pltpu.SemaphoreType.DMA((N_STEPS,)),
            pltpu.SemaphoreType.DMA((N_STEPS,)),
            pltpu.SemaphoreType.DMA((N_STEPS,)),
            pltpu.SemaphoreType.DMA((N_STEPS,)),
            pltpu.SemaphoreType.DMA((N_STEPS,)),
            pltpu.SemaphoreType.DMA((N_STEPS,)),
            pltpu.SemaphoreType.DMA((N_STEPS,)),
            pltpu.SemaphoreType.DMA((N_STEPS,)),
            pltpu.SemaphoreType.DMA((N_STEPS,)),
            pltpu.SemaphoreType.DMA((N_STEPS,)),
            pltpu.SemaphoreType.DMA((N_STEPS,)),
            pltpu.SemaphoreType.DMA((N_STEPS,)),
        ],
        compiler_params=pltpu.CompilerParams(
            collective_id=0,
            dimension_semantics=("arbitrary", "arbitrary"),
            vmem_limit_bytes=64 * 1024 * 1024,
        ),
    )(dy_qb, W)
